# baseline (speedup 1.0000x reference)
import functools

import jax
import jax.numpy as jnp
from jax import lax
from jax.experimental import pallas as pl
from jax.experimental.pallas import tpu as pltpu
from jax.experimental.pallas import tpu_sc as plsc

N_USERS = 100000
N_ITEMS = 100000
K = 128
B = 4096
U_MEAN = 3.5

NC = 2
NS = 16
L = 16
NW = NC * NS
BPW = B // NW
NCH = BPW // L
HALF = BPW // 2

_mesh = plsc.VectorSubcoreMesh(core_axis_name="c", subcore_axis_name="s")


@functools.partial(
    pl.kernel,
    mesh=_mesh,
    out_type=jax.ShapeDtypeStruct((B,), jnp.float32),
    scratch_types=[
        pltpu.VMEM((BPW,), jnp.int32),
        pltpu.VMEM((BPW,), jnp.int32),
        pltpu.VMEM((BPW,), jnp.float32),
        pltpu.VMEM((BPW,), jnp.float32),
        pltpu.VMEM((BPW,), jnp.float32),
        pltpu.VMEM((BPW, K), jnp.float32),
        pltpu.VMEM((BPW, K), jnp.float32),
        pltpu.VMEM((BPW,), jnp.float32),
        pltpu.SemaphoreType.DMA,
    ],
)
def _svdpp(u_h, i_h, bu_h, bi_h, pu_h, qit_h, ru_h, out_h,
           u_v, i_v, bu_v, bi_v, ru_v, pu_v, qt_v, res_v, sem):
    wid = lax.axis_index("s") * NC + lax.axis_index("c")
    base = wid * BPW

    cp_u = pltpu.async_copy(u_h.at[pl.ds(base, BPW)], u_v, sem)
    cp_i = pltpu.async_copy(i_h.at[pl.ds(base, BPW)], i_v, sem)
    cp_u.wait()
    cp_i.wait()

    cp_pu = pltpu.async_copy(pu_h.at[u_v], pu_v, sem)
    cp_qt = pltpu.async_copy(qit_h.at[i_v], qt_v, sem)
    cp_bu = pltpu.async_copy(bu_h.at[u_v], bu_v, sem)
    cp_bi = pltpu.async_copy(bi_h.at[i_v], bi_v, sem)
    cp_ru = pltpu.async_copy(ru_h.at[u_v], ru_v, sem)

    lane = lax.iota(jnp.int32, L)
    zero = jnp.zeros((L,), jnp.float32)
    perm8 = (lane + 8) & (L - 1)
    perm4 = (lane + 4) & (L - 1)

    cp_pu.wait()
    cp_qt.wait()
    cp_bu.wait()
    cp_bi.wait()
    cp_ru.wait()

    @plsc.parallel_loop(0, NCH)
    def group_body(g):
        sl = pl.ds(g * L, L)
        ruv = ru_v[sl]
        acc = zero
        for jj in range(L):
            j = g * L + jj
            rbc = lax.broadcast(ruv[jj], (L,))
            pa = zero
            for c in range(K // L):
                csl = pl.ds(c * L, L)
                pa = pa + (pu_v[j, csl] + rbc) * qt_v[j, csl]
            pa = pa + pa.at[perm8].get(mode="promise_in_bounds")
            pa = pa + pa.at[perm4].get(mode="promise_in_bounds")
            s = (pa[0] + pa[1]) + (pa[2] + pa[3])
            acc = jnp.where(lane == jj, lax.broadcast(s, (L,)), acc)
        res_v[sl] = bu_v[sl] + bi_v[sl] + U_MEAN + acc


    pltpu.sync_copy(res_v, out_h.at[pl.ds(base, BPW)])


def kernel(u, i, bu, bi, pu, qi, Ru):
    return _svdpp(
        u.astype(jnp.int32),
        i.astype(jnp.int32),
        bu,
        bi,
        pu,
        qi.T,
        Ru.reshape(-1),
    )


# speedup vs baseline: 1.1294x; 1.1294x over previous
import functools

import jax
import jax.numpy as jnp
from jax import lax
from jax.experimental import pallas as pl
from jax.experimental.pallas import tpu as pltpu
from jax.experimental.pallas import tpu_sc as plsc

N_USERS = 100000
N_ITEMS = 100000
K = 128
B = 4096
U_MEAN = 3.5

NC = 2
NS = 16
L = 16
NW = NC * NS
BPW = B // NW
NCH = BPW // L
HALF = BPW // 2

_mesh = plsc.VectorSubcoreMesh(core_axis_name="c", subcore_axis_name="s")


@functools.partial(
    pl.kernel,
    mesh=_mesh,
    out_type=jax.ShapeDtypeStruct((B,), jnp.float32),
    scratch_types=[
        pltpu.VMEM((BPW,), jnp.int32),
        pltpu.VMEM((BPW,), jnp.int32),
        pltpu.VMEM((BPW,), jnp.float32),
        pltpu.VMEM((BPW,), jnp.float32),
        pltpu.VMEM((BPW,), jnp.float32),
        pltpu.VMEM((BPW, K), jnp.float32),
        pltpu.VMEM((BPW, K), jnp.float32),
        pltpu.VMEM((BPW,), jnp.float32),
        pltpu.SemaphoreType.DMA,
    ],
)
def _svdpp(u_h, i_h, bu_h, bi_h, pu_h, qit_h, ru_h, out_h,
           u_v, i_v, bu_v, bi_v, ru_v, pu_v, qt_v, res_v, sem):
    wid = lax.axis_index("s") * NC + lax.axis_index("c")
    base = wid * BPW

    cp_u = pltpu.async_copy(u_h.at[pl.ds(base, BPW)], u_v, sem)
    cp_i = pltpu.async_copy(i_h.at[pl.ds(base, BPW)], i_v, sem)
    cp_u.wait()
    cp_i.wait()

    cp_pu = pltpu.async_copy(pu_h.at[u_v], pu_v, sem)
    cp_qt = pltpu.async_copy(qit_h.at[i_v], qt_v, sem)
    cp_bu = pltpu.async_copy(bu_h.at[u_v], bu_v, sem)
    cp_bi = pltpu.async_copy(bi_h.at[i_v], bi_v, sem)
    cp_ru = pltpu.async_copy(ru_h.at[u_v], ru_v, sem)

    lane = lax.iota(jnp.int32, L)
    zero = jnp.zeros((L,), jnp.float32)
    perm8 = (lane + 8) & (L - 1)

    cp_pu.wait()
    cp_qt.wait()
    cp_bu.wait()
    cp_bi.wait()
    cp_ru.wait()

    @plsc.parallel_loop(0, NCH, unroll=2)
    def group_body(g):
        sl = pl.ds(g * L, L)
        ruv = ru_v[sl]
        acc = zero
        for jj in range(L):
            j = g * L + jj
            rbc = lax.broadcast(ruv[jj], (L,))
            pa = zero
            for c in range(K // L):
                csl = pl.ds(c * L, L)
                pa = pa + (pu_v[j, csl] + rbc) * qt_v[j, csl]
            pa = pa + pa.at[perm8].get(mode="promise_in_bounds")
            s01 = pa[0] + pa[1]
            s23 = pa[2] + pa[3]
            s45 = pa[4] + pa[5]
            s67 = pa[6] + pa[7]
            s = (s01 + s23) + (s45 + s67)
            acc = jnp.where(lane == jj, lax.broadcast(s, (L,)), acc)
        res_v[sl] = bu_v[sl] + bi_v[sl] + U_MEAN + acc


    pltpu.sync_copy(res_v, out_h.at[pl.ds(base, BPW)])


def kernel(u, i, bu, bi, pu, qi, Ru):
    return _svdpp(
        u.astype(jnp.int32),
        i.astype(jnp.int32),
        bu,
        bi,
        pu,
        qi.T,
        Ru.reshape(-1),
    )


# speedup vs baseline: 1.1608x; 1.0278x over previous
import functools

import jax
import jax.numpy as jnp
from jax import lax
from jax.experimental import pallas as pl
from jax.experimental.pallas import tpu as pltpu
from jax.experimental.pallas import tpu_sc as plsc

N_USERS = 100000
N_ITEMS = 100000
K = 128
B = 4096
U_MEAN = 3.5

NC = 2
NS = 16
L = 16
NW = NC * NS
BPW = B // NW
NCH = BPW // L
HALF = BPW // 2

_mesh = plsc.VectorSubcoreMesh(core_axis_name="c", subcore_axis_name="s")


@functools.partial(
    pl.kernel,
    mesh=_mesh,
    out_type=jax.ShapeDtypeStruct((B,), jnp.float32),
    scratch_types=[
        pltpu.VMEM((BPW,), jnp.int32),
        pltpu.VMEM((BPW,), jnp.int32),
        pltpu.VMEM((BPW,), jnp.float32),
        pltpu.VMEM((BPW,), jnp.float32),
        pltpu.VMEM((BPW,), jnp.float32),
        pltpu.VMEM((BPW, K), jnp.float32),
        pltpu.VMEM((BPW, K), jnp.float32),
        pltpu.VMEM((BPW,), jnp.float32),
        pltpu.SemaphoreType.DMA,
    ],
)
def _svdpp(u_h, i_h, bu_h, bi_h, pu_h, qit_h, ru_h, out_h,
           u_v, i_v, bu_v, bi_v, ru_v, pu_v, qt_v, res_v, sem):
    wid = lax.axis_index("s") * NC + lax.axis_index("c")
    base = wid * BPW

    cp_u = pltpu.async_copy(u_h.at[pl.ds(base, BPW)], u_v, sem)
    cp_i = pltpu.async_copy(i_h.at[pl.ds(base, BPW)], i_v, sem)
    cp_u.wait()
    cp_i.wait()

    cp_pu = pltpu.async_copy(pu_h.at[u_v], pu_v, sem)
    cp_qt = pltpu.async_copy(qit_h.at[i_v], qt_v, sem)
    cp_bu = pltpu.async_copy(bu_h.at[u_v], bu_v, sem)
    cp_bi = pltpu.async_copy(bi_h.at[i_v], bi_v, sem)
    cp_ru = pltpu.async_copy(ru_h.at[u_v], ru_v, sem)

    lane = lax.iota(jnp.int32, L)
    zero = jnp.zeros((L,), jnp.float32)
    perm8 = (lane + 8) & (L - 1)

    cp_pu.wait()
    cp_qt.wait()
    cp_bu.wait()
    cp_bi.wait()
    cp_ru.wait()

    @plsc.parallel_loop(0, NCH)
    def group_body(g):
        sl = pl.ds(g * L, L)
        ruv = ru_v[sl]
        acc = zero
        for jj in range(L):
            j = g * L + jj
            rbc = lax.broadcast(ruv[jj], (L,))
            pa = zero
            for c in range(K // L):
                csl = pl.ds(c * L, L)
                pa = pa + (pu_v[j, csl] + rbc) * qt_v[j, csl]
            pa = pa + pa.at[perm8].get(mode="promise_in_bounds")
            s01 = pa[0] + pa[1]
            s23 = pa[2] + pa[3]
            s45 = pa[4] + pa[5]
            s67 = pa[6] + pa[7]
            s = (s01 + s23) + (s45 + s67)
            acc = jnp.where(lane == jj, lax.broadcast(s, (L,)), acc)
        res_v[sl] = bu_v[sl] + bi_v[sl] + U_MEAN + acc


    pltpu.sync_copy(res_v, out_h.at[pl.ds(base, BPW)])


def kernel(u, i, bu, bi, pu, qi, Ru):
    return _svdpp(
        u.astype(jnp.int32),
        i.astype(jnp.int32),
        bu,
        bi,
        pu,
        qi.T,
        Ru.reshape(-1),
    )
